# baseline (speedup 1.0000x reference)
import functools

import jax
import jax.numpy as jnp
import numpy as np
from jax import lax
from jax.experimental import pallas as pl
from jax.experimental.pallas import tpu as pltpu
from jax.experimental.pallas import tpu_sc as plsc

N_ROWS = 1_000_000
DIM = 128
VOCAB = 324
BLK = 128
NBLK = N_ROWS // BLK
TAIL = N_ROWS - NBLK * BLK
NC, NS = 2, 16
NW = NC * NS
NB_LO = NBLK // NW
N_EXTRA = NBLK - NB_LO * NW
NB_HI = NB_LO + 1
CB = 2
NCHUNK = NB_LO // CB
N_OUTER = NCHUNK // 2

_mesh = plsc.VectorSubcoreMesh(core_axis_name="c", subcore_axis_name="s")


@functools.partial(
    pl.kernel,
    mesh=_mesh,
    out_type=jax.ShapeDtypeStruct((N_ROWS, DIM), jnp.float32),
    scratch_types=[
        pltpu.VMEM_SHARED((VOCAB, DIM), jnp.float32),
        pltpu.VMEM((NB_HI * BLK,), jnp.int32),
        pltpu.VMEM((2, CB * BLK, DIM), jnp.float32),
        pltpu.VMEM((TAIL,), jnp.int32),
        pltpu.VMEM((TAIL, DIM), jnp.float32),
        pltpu.SemaphoreType.DMA,
        pltpu.SemaphoreType.DMA,
        pltpu.SemaphoreType.DMA,
        pltpu.SemaphoreType.DMA,
    ],
)
def _gather_kernel(emb_hbm, idx1d_hbm, out_hbm,
                   table_sh, idx_v, rows_v, tidx_v, trows_v,
                   gsem0, gsem1, ssem0, ssem1):
    wid = lax.axis_index("s") * NC + lax.axis_index("c")
    gsems = (gsem0, gsem1)
    ssems = (ssem0, ssem1)

    @pl.when(lax.axis_index("s") == 0)
    def _():
        pltpu.sync_copy(emb_hbm, table_sh)
    plsc.subcore_barrier()

    start = wid * NB_LO + lax.min(wid, N_EXTRA)
    n_blk = NB_LO + lax.convert_element_type(wid < N_EXTRA, jnp.int32)

    @pl.when(start + NB_HI <= NBLK)
    def _():
        pltpu.sync_copy(idx1d_hbm.at[pl.ds(start * BLK, NB_HI * BLK)], idx_v)

    @pl.when(start + NB_HI > NBLK)
    def _():
        pltpu.sync_copy(
            idx1d_hbm.at[pl.ds(start * BLK, NB_LO * BLK)],
            idx_v.at[pl.ds(0, NB_LO * BLK)],
        )

    def idx_row(k):
        return idx_v.at[pl.ds(k * BLK, BLK)]

    def fire_chunk(c, b):
        for h in range(CB):
            pltpu.async_copy(
                table_sh.at[idx_row(c * CB + h)],
                rows_v.at[b, pl.ds(h * BLK, BLK)],
                gsems[b],
            )

    def wait_gathers(b, n):
        for _ in range(n):
            pltpu.make_async_copy(
                table_sh.at[idx_row(0)], rows_v.at[b, pl.ds(0, BLK)], gsems[b]
            ).wait()

    fire_chunk(0, 0)

    def step(c, b):
        @pl.when(c + 1 < NCHUNK)
        def _():
            @pl.when(c >= 1)
            def _():
                pltpu.make_async_copy(
                    rows_v.at[1 - b], out_hbm.at[pl.ds(0, CB * BLK)],
                    ssems[1 - b],
                ).wait()
            fire_chunk(c + 1, 1 - b)

        wait_gathers(b, CB)
        pltpu.async_copy(
            rows_v.at[b],
            out_hbm.at[pl.ds((start + c * CB) * BLK, CB * BLK)],
            ssems[b],
        )

    def body(ii, carry):
        c2 = ii * 2
        step(c2, 0)
        step(c2 + 1, 1)
        return carry

    lax.fori_loop(0, N_OUTER, body, 0)

    @pl.when(n_blk > NB_LO)
    def _():
        pltpu.make_async_copy(
            rows_v.at[0], out_hbm.at[pl.ds(0, CB * BLK)], ssems[0]
        ).wait()
        pltpu.async_copy(
            table_sh.at[idx_row(NB_LO)], rows_v.at[0, pl.ds(0, BLK)], gsems[0]
        )
        wait_gathers(0, 1)
        pltpu.async_copy(
            rows_v.at[0, pl.ds(0, BLK)],
            out_hbm.at[pl.ds((start + NB_LO) * BLK, BLK)],
            ssems[0],
        )
        pltpu.make_async_copy(
            rows_v.at[0, pl.ds(0, BLK)], out_hbm.at[pl.ds(0, BLK)], ssems[0]
        ).wait()
        pltpu.make_async_copy(
            rows_v.at[1], out_hbm.at[pl.ds(0, CB * BLK)], ssems[1]
        ).wait()

    @pl.when(n_blk == NB_LO)
    def _():
        pltpu.make_async_copy(
            rows_v.at[0], out_hbm.at[pl.ds(0, CB * BLK)], ssems[0]
        ).wait()
        pltpu.make_async_copy(
            rows_v.at[1], out_hbm.at[pl.ds(0, CB * BLK)], ssems[1]
        ).wait()

    @pl.when(wid == 0)
    def _():
        pltpu.sync_copy(idx1d_hbm.at[pl.ds(NBLK * BLK, TAIL)], tidx_v)
        pltpu.async_copy(table_sh.at[tidx_v], trows_v, gsems[0]).wait()
        pltpu.sync_copy(trows_v, out_hbm.at[pl.ds(NBLK * BLK, TAIL)])


def kernel(indices, embeddings):
    return _gather_kernel(embeddings, indices.astype(jnp.int32))


# speedup vs baseline: 1.0121x; 1.0121x over previous
import functools

import jax
import jax.numpy as jnp
import numpy as np
from jax import lax
from jax.experimental import pallas as pl
from jax.experimental.pallas import tpu as pltpu
from jax.experimental.pallas import tpu_sc as plsc

N_ROWS = 1_000_000
DIM = 128
VOCAB = 324
BLK = 128
NBLK = N_ROWS // BLK
TAIL = N_ROWS - NBLK * BLK
NC, NS = 2, 16
NW = NC * NS
NB_LO = NBLK // NW
N_EXTRA = NBLK - NB_LO * NW
NB_HI = NB_LO + 1
NBUF = 4
N_OUTER = NB_LO // NBUF

_mesh = plsc.VectorSubcoreMesh(core_axis_name="c", subcore_axis_name="s")


@functools.partial(
    pl.kernel,
    mesh=_mesh,
    out_type=jax.ShapeDtypeStruct((N_ROWS, DIM), jnp.float32),
    scratch_types=[
        pltpu.VMEM_SHARED((VOCAB, DIM), jnp.float32),
        pltpu.VMEM((NB_HI * BLK,), jnp.int32),
        pltpu.VMEM((NBUF, BLK, DIM), jnp.float32),
        pltpu.VMEM((TAIL,), jnp.int32),
        pltpu.VMEM((TAIL, DIM), jnp.float32),
        pltpu.SemaphoreType.DMA,
        pltpu.SemaphoreType.DMA,
        pltpu.SemaphoreType.DMA,
        pltpu.SemaphoreType.DMA,
        pltpu.SemaphoreType.DMA,
        pltpu.SemaphoreType.DMA,
        pltpu.SemaphoreType.DMA,
        pltpu.SemaphoreType.DMA,
    ],
)
def _gather_kernel(emb_hbm, idx1d_hbm, out_hbm,
                   table_sh, idx_v, rows_v, tidx_v, trows_v,
                   gsem0, gsem1, gsem2, gsem3, ssem0, ssem1, ssem2, ssem3):
    wid = lax.axis_index("s") * NC + lax.axis_index("c")
    gsems = (gsem0, gsem1, gsem2, gsem3)
    ssems = (ssem0, ssem1, ssem2, ssem3)

    @pl.when(lax.axis_index("s") == 0)
    def _():
        pltpu.sync_copy(emb_hbm, table_sh)
    plsc.subcore_barrier()

    start = wid * NB_LO + lax.min(wid, N_EXTRA)
    n_blk = NB_LO + lax.convert_element_type(wid < N_EXTRA, jnp.int32)

    @pl.when(start + NB_HI <= NBLK)
    def _():
        pltpu.sync_copy(idx1d_hbm.at[pl.ds(start * BLK, NB_HI * BLK)], idx_v)

    @pl.when(start + NB_HI > NBLK)
    def _():
        pltpu.sync_copy(
            idx1d_hbm.at[pl.ds(start * BLK, NB_LO * BLK)],
            idx_v.at[pl.ds(0, NB_LO * BLK)],
        )

    def idx_row(k):
        return idx_v.at[pl.ds(k * BLK, BLK)]

    pltpu.async_copy(table_sh.at[idx_row(0)], rows_v.at[0], gsems[0])
    pltpu.async_copy(table_sh.at[idx_row(1)], rows_v.at[1], gsems[1])

    def step(i, b):
        nx = i + 2
        bx = (b + 2) % NBUF

        @pl.when(nx < n_blk)
        def _():
            @pl.when(nx >= NBUF)
            def _():
                pltpu.make_async_copy(
                    rows_v.at[bx], out_hbm.at[pl.ds(0, BLK)], ssems[bx]
                ).wait()
            pltpu.async_copy(table_sh.at[idx_row(nx)], rows_v.at[bx], gsems[bx])

        pltpu.make_async_copy(
            table_sh.at[idx_row(0)], rows_v.at[b], gsems[b]
        ).wait()
        pltpu.async_copy(
            rows_v.at[b], out_hbm.at[pl.ds((start + i) * BLK, BLK)], ssems[b]
        )

    def body(ii, carry):
        i4 = ii * NBUF
        for db in range(NBUF):
            step(i4 + db, db)
        return carry

    lax.fori_loop(0, N_OUTER, body, 0)

    @pl.when(n_blk > NB_LO)
    def _():
        pltpu.make_async_copy(
            table_sh.at[idx_row(0)], rows_v.at[0], gsems[0]
        ).wait()
        pltpu.async_copy(
            rows_v.at[0], out_hbm.at[pl.ds((start + NB_LO) * BLK, BLK)], ssems[0]
        )

    for b in range(NBUF):
        pltpu.make_async_copy(
            rows_v.at[b], out_hbm.at[pl.ds(0, BLK)], ssems[b]
        ).wait()

    @pl.when(wid == 0)
    def _():
        pltpu.sync_copy(idx1d_hbm.at[pl.ds(NBLK * BLK, TAIL)], tidx_v)
        pltpu.async_copy(table_sh.at[tidx_v], trows_v, gsems[0]).wait()
        pltpu.sync_copy(trows_v, out_hbm.at[pl.ds(NBLK * BLK, TAIL)])


def kernel(indices, embeddings):
    return _gather_kernel(embeddings, indices.astype(jnp.int32))
